# baseline (speedup 1.0000x reference)
import functools

import jax
import jax.numpy as jnp
from jax import lax
from jax.experimental import pallas as pl
from jax.experimental.pallas import tpu as pltpu
from jax.experimental.pallas import tpu_sc as plsc

T = 2048
D = 768
DFF = 3072
E = 8
BLK = 256
NB = 24
RP = NB * BLK
NW = 32
TPW = T // NW
RCH = 512


def _route_body(x_ref, gw_ref, gb_ref,
                logits_ref, pos0_ref, pos1_ref, w0_ref, w1_ref,
                beid_ref):
    xv = x_ref[...]
    logits = lax.dot_general(xv, gw_ref[...], (((1,), (1,)), ((), ())),
                             preferred_element_type=jnp.float32) + gb_ref[...]
    logits_ref[...] = logits

    lane = lax.broadcasted_iota(jnp.int32, (T, E), 1)
    m1 = jnp.max(logits, axis=-1, keepdims=True)
    a1 = jnp.min(jnp.where(logits == m1, lane, E), axis=-1, keepdims=True)
    masked = jnp.where(lane == a1, -jnp.inf, logits)
    m2 = jnp.max(masked, axis=-1, keepdims=True)
    a2 = jnp.min(jnp.where(masked == m2, lane, E), axis=-1, keepdims=True)
    w1 = 1.0 / (1.0 + jnp.exp(m2 - m1))
    w2 = 1.0 - w1
    w0_ref[...] = jnp.broadcast_to(w1, (T, 16))
    w1_ref[...] = jnp.broadcast_to(w2, (T, 16))

    oh1 = jnp.where(lane == a1, 1.0, 0.0)
    oh2 = jnp.where(lane == a2, 1.0, 0.0)

    cnt1 = jnp.sum(oh1, axis=0, keepdims=True)
    cnt2 = jnp.sum(oh2, axis=0, keepdims=True)
    counts = cnt1 + cnt2
    pc = jnp.floor((counts + (BLK - 1.0)) / BLK) * BLK
    ei = lax.broadcasted_iota(jnp.int32, (E, E), 0)
    ej = lax.broadcasted_iota(jnp.int32, (E, E), 1)
    excl = jnp.where(ei < ej, 1.0, 0.0)
    starts = lax.dot_general(pc, excl, (((1,), (0,)), ((), ())),
                             preferred_element_type=jnp.float32)

    bii = lax.broadcasted_iota(jnp.int32, (1, 32), 1)
    biota = bii.astype(jnp.float32)
    eid = jnp.zeros((1, 32), jnp.float32) - 1.0
    for e in range(E):
        sb_e = starts[0, e] / float(BLK)
        eid = eid + jnp.where(biota >= sb_e, 1.0, 0.0)
    nbu = jnp.sum(pc) / float(BLK)
    eseg = jnp.where(bii == NB, nbu, eid)
    prev = jnp.concatenate([eid[:, :1], eid[:, :31]], axis=1)
    first = jnp.where(eid != prev, 1.0, 0.0)
    first = jnp.where(bii == 0, 1.0, first)
    mi = lax.broadcasted_iota(jnp.int32, (32, 32), 0)
    mj = lax.broadcasted_iota(jnp.int32, (32, 32), 1)
    mle = jnp.where(mi <= mj, 1.0, 0.0)
    runidx = lax.dot_general(first, mle, (((1,), (0,)), ((), ())),
                             preferred_element_type=jnp.float32) - 1.0
    slot = runidx - 2.0 * jnp.floor(runidx * 0.5)
    pfe = jnp.zeros((1, 32), jnp.float32) - 1.0
    for e in range(E - 1, -1, -1):
        nonempty = counts[0, e] > 0.0
        cond = jnp.logical_and(jnp.full((1, 32), nonempty), float(e) > eid)
        pfe = jnp.where(cond, float(e), pfe)
    beid_ref[...] = jnp.concatenate(
        [eseg, slot, first, pfe], axis=1).astype(jnp.int32)

    ri = lax.broadcasted_iota(jnp.int32, (RCH, RCH), 0)
    rj = lax.broadcasted_iota(jnp.int32, (RCH, RCH), 1)
    tri = jnp.where(rj < ri, 1.0, 0.0)
    startsb = jnp.broadcast_to(starts, (RCH, E))
    carry1 = jnp.zeros((1, E), jnp.float32)
    carry2 = cnt1
    for c in range(T // RCH):
        sl = pl.ds(c * RCH, RCH)
        ohc1 = oh1[c * RCH:(c + 1) * RCH, :]
        ohc2 = oh2[c * RCH:(c + 1) * RCH, :]
        exc1 = lax.dot_general(tri, ohc1, (((1,), (0,)), ((), ())),
                               preferred_element_type=jnp.float32) + carry1
        exc2 = lax.dot_general(tri, ohc2, (((1,), (0,)), ((), ())),
                               preferred_element_type=jnp.float32) + carry2
        p0 = jnp.sum(ohc1 * (exc1 + startsb), axis=-1, keepdims=True)
        p1 = jnp.sum(ohc2 * (exc2 + startsb), axis=-1, keepdims=True)
        pos0_ref[sl, :] = p0.astype(jnp.int32)
        pos1_ref[sl, :] = p1.astype(jnp.int32)
        carry1 = carry1 + jnp.sum(ohc1, axis=0, keepdims=True)
        carry2 = carry2 + jnp.sum(ohc2, axis=0, keepdims=True)

def _route(tokens, gate_W, gb):
    return pl.pallas_call(
        _route_body,
        in_specs=[
            pl.BlockSpec((T, D), lambda: (0, 0)),
            pl.BlockSpec((E, D), lambda: (0, 0)),
            pl.BlockSpec((1, E), lambda: (0, 0)),
        ],
        out_specs=[
            pl.BlockSpec((T, E), lambda: (0, 0)),
            pl.BlockSpec((T, 1), lambda: (0, 0)),
            pl.BlockSpec((T, 1), lambda: (0, 0)),
            pl.BlockSpec((T, 16), lambda: (0, 0)),
            pl.BlockSpec((T, 16), lambda: (0, 0)),
            pl.BlockSpec((1, 128), lambda: (0, 0)),
        ],
        out_shape=[
            jax.ShapeDtypeStruct((T, E), jnp.float32),
            jax.ShapeDtypeStruct((T, 1), jnp.int32),
            jax.ShapeDtypeStruct((T, 1), jnp.int32),
            jax.ShapeDtypeStruct((T, 16), jnp.float32),
            jax.ShapeDtypeStruct((T, 16), jnp.float32),
            jax.ShapeDtypeStruct((1, 128), jnp.int32),
        ],
    )(tokens, gate_W, gb)


def _ffn_body(sc_ref, xg_ref, wx_hbm, vx_hbm, wo_hbm, og_ref,
              wxb, vxb, wob, sem0, sem1):
    i = pl.program_id(0)
    eidv = sc_ref[i]
    nbu = sc_ref[NB]
    slot = sc_ref[32 + i]
    first = sc_ref[64 + i]
    pfe = sc_ref[96 + i]

    def start(e, s, sm):
        pltpu.make_async_copy(wx_hbm.at[e], wxb.at[s], sm).start()
        pltpu.make_async_copy(vx_hbm.at[e], vxb.at[s], sm).start()
        pltpu.make_async_copy(wo_hbm.at[e], wob.at[s], sm).start()

    def wait(s, sm):
        pltpu.make_async_copy(wx_hbm.at[0], wxb.at[s], sm).wait()
        pltpu.make_async_copy(vx_hbm.at[0], vxb.at[s], sm).wait()
        pltpu.make_async_copy(wo_hbm.at[0], wob.at[s], sm).wait()

    @pl.when(i == 0)
    def _prologue():
        start(eidv, 0, sem0)

        @pl.when(pfe >= 0)
        def _():
            start(pfe, 1, sem1)

        wait(0, sem0)

    @pl.when(jnp.logical_and(i > 0, first == 1))
    def _run_start():
        @pl.when(slot == 0)
        def _():
            wait(0, sem0)

            @pl.when(pfe >= 0)
            def _():
                start(pfe, 1, sem1)

        @pl.when(slot == 1)
        def _():
            wait(1, sem1)

            @pl.when(pfe >= 0)
            def _():
                start(pfe, 0, sem0)

    @pl.when(i < nbu)
    def _compute():
        xv = xg_ref[...]
        a = lax.dot_general(xv, wxb[slot], (((1,), (1,)), ((), ())),
                            preferred_element_type=jnp.float32)
        b = lax.dot_general(xv, vxb[slot], (((1,), (1,)), ((), ())),
                            preferred_element_type=jnp.float32)
        h = a * jax.nn.sigmoid(a) * b
        og_ref[...] = lax.dot_general(h, wob[slot], (((1,), (1,)), ((), ())),
                                      preferred_element_type=jnp.float32)


def _ffn(blk_sched, Xg, Wx, Vx, Wo):
    grid_spec = pltpu.PrefetchScalarGridSpec(
        num_scalar_prefetch=1,
        grid=(NB,),
        in_specs=[
            pl.BlockSpec((BLK, D), lambda i, sc: (i, 0)),
            pl.BlockSpec(memory_space=pl.ANY),
            pl.BlockSpec(memory_space=pl.ANY),
            pl.BlockSpec(memory_space=pl.ANY),
        ],
        out_specs=pl.BlockSpec((BLK, D), lambda i, sc: (i, 0)),
        scratch_shapes=[
            pltpu.VMEM((2, DFF, D), jnp.float32),
            pltpu.VMEM((2, DFF, D), jnp.float32),
            pltpu.VMEM((2, D, DFF), jnp.float32),
            pltpu.SemaphoreType.DMA,
            pltpu.SemaphoreType.DMA,
        ],
    )
    return pl.pallas_call(
        _ffn_body,
        grid_spec=grid_spec,
        out_shape=jax.ShapeDtypeStruct((RP, D), jnp.float32),
        compiler_params=pltpu.CompilerParams(
            dimension_semantics=("arbitrary",),
            vmem_limit_bytes=100 * 1024 * 1024),
    )(blk_sched, Xg, Wx, Vx, Wo)


def _sc_mesh():
    return plsc.VectorSubcoreMesh(core_axis_name="c", subcore_axis_name="s",
                                  num_cores=2, num_subcores=16)


def _dispatch(tokens, pos0, pos1):
    def body(x_hbm, p0_hbm, p1_hbm, out_hbm, i0, i1, rows_v, sem):
        wid = lax.axis_index("s") * 2 + lax.axis_index("c")
        base = wid * TPW
        a = pltpu.async_copy(p0_hbm.at[pl.ds(base, TPW)], i0, sem)
        b = pltpu.async_copy(p1_hbm.at[pl.ds(base, TPW)], i1, sem)
        c = pltpu.async_copy(x_hbm.at[pl.ds(base, TPW)], rows_v, sem)
        a.wait()
        b.wait()
        c.wait()
        c1 = pltpu.async_copy(rows_v, out_hbm.at[i0], sem)
        c2 = pltpu.async_copy(rows_v, out_hbm.at[i1], sem)
        c1.wait()
        c2.wait()

    f = pl.kernel(
        body,
        out_type=jax.ShapeDtypeStruct((RP, D), jnp.float32),
        mesh=_sc_mesh(),
        scratch_types=[
            pltpu.VMEM((TPW,), jnp.int32),
            pltpu.VMEM((TPW,), jnp.int32),
            pltpu.VMEM((TPW, D), jnp.float32),
            pltpu.SemaphoreType.DMA,
        ],
    )
    return f(tokens, pos0, pos1)


def _combine(Og, pos0, pos1, w0x, w1x):
    def body(og_hbm, p0_hbm, p1_hbm, w0_hbm, w1_hbm, out_hbm,
             i0, i1, av, bv, wv0, wv1, sem):
        wid = lax.axis_index("s") * 2 + lax.axis_index("c")
        base = wid * TPW
        a = pltpu.async_copy(p0_hbm.at[pl.ds(base, TPW)], i0, sem)
        b = pltpu.async_copy(p1_hbm.at[pl.ds(base, TPW)], i1, sem)
        a.wait()
        b.wait()
        c1 = pltpu.async_copy(og_hbm.at[i0], av, sem)
        c2 = pltpu.async_copy(og_hbm.at[i1], bv, sem)
        pltpu.sync_copy(w0_hbm.at[pl.ds(base, TPW)], wv0)
        pltpu.sync_copy(w1_hbm.at[pl.ds(base, TPW)], wv1)
        c1.wait()
        c2.wait()

        def row(r, _):
            wa = wv0[r, :]
            wb = wv1[r, :]
            for c in range(D // 16):
                s = pl.ds(c * 16, 16)
                av[r, s] = av[r, s] * wa + bv[r, s] * wb
            return 0

        lax.fori_loop(0, TPW, row, 0)
        pltpu.sync_copy(av, out_hbm.at[pl.ds(base, TPW)])

    f = pl.kernel(
        body,
        out_type=jax.ShapeDtypeStruct((T, D), jnp.float32),
        mesh=_sc_mesh(),
        scratch_types=[
            pltpu.VMEM((TPW,), jnp.int32),
            pltpu.VMEM((TPW,), jnp.int32),
            pltpu.VMEM((TPW, D), jnp.float32),
            pltpu.VMEM((TPW, D), jnp.float32),
            pltpu.VMEM((TPW, 16), jnp.float32),
            pltpu.VMEM((TPW, 16), jnp.float32),
            pltpu.SemaphoreType.DMA,
        ],
    )
    return f(Og, pos0, pos1, w0x, w1x)


@functools.partial(jax.jit, static_argnames=())
def kernel(x, gate_W, gate_b, Wx, Vx, Wo):
    bsz, seq, d = x.shape
    tokens = x.reshape(-1, d)
    gb = gate_b.reshape(1, E)
    (logits, pos0c, pos1c, w0x, w1x, beid_row) = _route(tokens, gate_W, gb)
    pos0 = pos0c.reshape(T)
    pos1 = pos1c.reshape(T)
    blk_sched = beid_row[0]
    Xg = _dispatch(tokens, pos0, pos1)
    Og = _ffn(blk_sched, Xg, Wx, Vx, Wo)
    out = _combine(Og, pos0, pos1, w0x, w1x)
    return out.reshape(bsz, seq, d), logits


# speedup vs baseline: 1.0174x; 1.0174x over previous
import functools

import jax
import jax.numpy as jnp
from jax import lax
from jax.experimental import pallas as pl
from jax.experimental.pallas import tpu as pltpu
from jax.experimental.pallas import tpu_sc as plsc

T = 2048
D = 768
DFF = 3072
E = 8
BLK = 256
NB = 24
RP = NB * BLK
NW = 32
TPW = T // NW
RCH = 512


def _route_body(x_ref, gw_ref, gb_ref,
                logits_ref, pos0_ref, pos1_ref, w0_ref, w1_ref,
                beid_ref):
    xv = x_ref[...]
    logits = lax.dot_general(xv, gw_ref[...], (((1,), (1,)), ((), ())),
                             preferred_element_type=jnp.float32) + gb_ref[...]
    logits_ref[...] = logits

    lane = lax.broadcasted_iota(jnp.int32, (T, E), 1)
    m1 = jnp.max(logits, axis=-1, keepdims=True)
    a1 = jnp.min(jnp.where(logits == m1, lane, E), axis=-1, keepdims=True)
    masked = jnp.where(lane == a1, -jnp.inf, logits)
    m2 = jnp.max(masked, axis=-1, keepdims=True)
    a2 = jnp.min(jnp.where(masked == m2, lane, E), axis=-1, keepdims=True)
    w1 = 1.0 / (1.0 + jnp.exp(m2 - m1))
    w2 = 1.0 - w1
    w0_ref[...] = jnp.broadcast_to(w1, (T, 16))
    w1_ref[...] = jnp.broadcast_to(w2, (T, 16))

    oh1 = jnp.where(lane == a1, 1.0, 0.0)
    oh2 = jnp.where(lane == a2, 1.0, 0.0)

    cnt1 = jnp.sum(oh1, axis=0, keepdims=True)
    cnt2 = jnp.sum(oh2, axis=0, keepdims=True)
    counts = cnt1 + cnt2
    pc = jnp.floor((counts + (BLK - 1.0)) / BLK) * BLK
    ei = lax.broadcasted_iota(jnp.int32, (E, E), 0)
    ej = lax.broadcasted_iota(jnp.int32, (E, E), 1)
    excl = jnp.where(ei < ej, 1.0, 0.0)
    starts = lax.dot_general(pc, excl, (((1,), (0,)), ((), ())),
                             preferred_element_type=jnp.float32)

    bii = lax.broadcasted_iota(jnp.int32, (1, 32), 1)
    biota = bii.astype(jnp.float32)
    eid = jnp.zeros((1, 32), jnp.float32)
    for e in range(E):
        sb_e = starts[0, e] / float(BLK)
        nonempty_e = counts[0, e] > 0.0
        cond = jnp.logical_and(jnp.full((1, 32), nonempty_e), biota >= sb_e)
        eid = jnp.where(cond, float(e), eid)
    nbu = jnp.sum(pc) / float(BLK)
    eseg = jnp.where(bii == NB, nbu, eid)
    prev = jnp.concatenate([eid[:, :1], eid[:, :31]], axis=1)
    first = jnp.where(eid != prev, 1.0, 0.0)
    first = jnp.where(bii == 0, 1.0, first)
    mi = lax.broadcasted_iota(jnp.int32, (32, 32), 0)
    mj = lax.broadcasted_iota(jnp.int32, (32, 32), 1)
    mle = jnp.where(mi <= mj, 1.0, 0.0)
    runidx = lax.dot_general(first, mle, (((1,), (0,)), ((), ())),
                             preferred_element_type=jnp.float32) - 1.0
    slot = runidx - 2.0 * jnp.floor(runidx * 0.5)
    pfe = jnp.zeros((1, 32), jnp.float32) - 1.0
    for e in range(E - 1, -1, -1):
        nonempty = counts[0, e] > 0.0
        cond = jnp.logical_and(jnp.full((1, 32), nonempty), float(e) > eid)
        pfe = jnp.where(cond, float(e), pfe)
    beid_ref[...] = jnp.concatenate(
        [eseg, slot, first, pfe], axis=1).astype(jnp.int32)

    ri = lax.broadcasted_iota(jnp.int32, (RCH, RCH), 0)
    rj = lax.broadcasted_iota(jnp.int32, (RCH, RCH), 1)
    tri = jnp.where(rj < ri, 1.0, 0.0)
    startsb = jnp.broadcast_to(starts, (RCH, E))
    carry1 = jnp.zeros((1, E), jnp.float32)
    carry2 = cnt1
    for c in range(T // RCH):
        sl = pl.ds(c * RCH, RCH)
        ohc1 = oh1[c * RCH:(c + 1) * RCH, :]
        ohc2 = oh2[c * RCH:(c + 1) * RCH, :]
        exc1 = lax.dot_general(tri, ohc1, (((1,), (0,)), ((), ())),
                               preferred_element_type=jnp.float32) + carry1
        exc2 = lax.dot_general(tri, ohc2, (((1,), (0,)), ((), ())),
                               preferred_element_type=jnp.float32) + carry2
        p0 = jnp.sum(ohc1 * (exc1 + startsb), axis=-1, keepdims=True)
        p1 = jnp.sum(ohc2 * (exc2 + startsb), axis=-1, keepdims=True)
        pos0_ref[sl, :] = p0.astype(jnp.int32)
        pos1_ref[sl, :] = p1.astype(jnp.int32)
        carry1 = carry1 + jnp.sum(ohc1, axis=0, keepdims=True)
        carry2 = carry2 + jnp.sum(ohc2, axis=0, keepdims=True)

def _route(tokens, gate_W, gb):
    return pl.pallas_call(
        _route_body,
        in_specs=[
            pl.BlockSpec((T, D), lambda: (0, 0)),
            pl.BlockSpec((E, D), lambda: (0, 0)),
            pl.BlockSpec((1, E), lambda: (0, 0)),
        ],
        out_specs=[
            pl.BlockSpec((T, E), lambda: (0, 0)),
            pl.BlockSpec((T, 1), lambda: (0, 0)),
            pl.BlockSpec((T, 1), lambda: (0, 0)),
            pl.BlockSpec((T, 16), lambda: (0, 0)),
            pl.BlockSpec((T, 16), lambda: (0, 0)),
            pl.BlockSpec((1, 128), lambda: (0, 0)),
        ],
        out_shape=[
            jax.ShapeDtypeStruct((T, E), jnp.float32),
            jax.ShapeDtypeStruct((T, 1), jnp.int32),
            jax.ShapeDtypeStruct((T, 1), jnp.int32),
            jax.ShapeDtypeStruct((T, 16), jnp.float32),
            jax.ShapeDtypeStruct((T, 16), jnp.float32),
            jax.ShapeDtypeStruct((1, 128), jnp.int32),
        ],
    )(tokens, gate_W, gb)


def _ffn_body(sc_ref, xg_ref, wx_hbm, vx_hbm, wo_hbm, og_ref,
              wxb, vxb, wob, sem0, sem1):
    i = pl.program_id(0)
    eidv = sc_ref[i]
    nbu = sc_ref[NB]
    slot = sc_ref[32 + i]
    first = sc_ref[64 + i]
    pfe = sc_ref[96 + i]

    def start(e, s, sm):
        pltpu.make_async_copy(wx_hbm.at[e], wxb.at[s], sm).start()
        pltpu.make_async_copy(vx_hbm.at[e], vxb.at[s], sm).start()
        pltpu.make_async_copy(wo_hbm.at[e], wob.at[s], sm).start()

    def wait(s, sm):
        pltpu.make_async_copy(wx_hbm.at[0], wxb.at[s], sm).wait()
        pltpu.make_async_copy(vx_hbm.at[0], vxb.at[s], sm).wait()
        pltpu.make_async_copy(wo_hbm.at[0], wob.at[s], sm).wait()

    @pl.when(i == 0)
    def _prologue():
        start(eidv, 0, sem0)

        @pl.when(pfe >= 0)
        def _():
            start(pfe, 1, sem1)

        wait(0, sem0)

    @pl.when(jnp.logical_and(i > 0, first == 1))
    def _run_start():
        @pl.when(slot == 0)
        def _():
            wait(0, sem0)

            @pl.when(pfe >= 0)
            def _():
                start(pfe, 1, sem1)

        @pl.when(slot == 1)
        def _():
            wait(1, sem1)

            @pl.when(pfe >= 0)
            def _():
                start(pfe, 0, sem0)

    @pl.when(i < nbu)
    def _compute():
        xv = xg_ref[...]
        a = lax.dot_general(xv, wxb[slot], (((1,), (1,)), ((), ())),
                            preferred_element_type=jnp.float32)
        b = lax.dot_general(xv, vxb[slot], (((1,), (1,)), ((), ())),
                            preferred_element_type=jnp.float32)
        h = a * jax.nn.sigmoid(a) * b
        og_ref[...] = lax.dot_general(h, wob[slot], (((1,), (1,)), ((), ())),
                                      preferred_element_type=jnp.float32)


def _ffn(blk_sched, Xg, Wx, Vx, Wo):
    grid_spec = pltpu.PrefetchScalarGridSpec(
        num_scalar_prefetch=1,
        grid=(NB,),
        in_specs=[
            pl.BlockSpec((BLK, D), lambda i, sc: (i, 0)),
            pl.BlockSpec(memory_space=pl.ANY),
            pl.BlockSpec(memory_space=pl.ANY),
            pl.BlockSpec(memory_space=pl.ANY),
        ],
        out_specs=pl.BlockSpec((BLK, D), lambda i, sc: (i, 0)),
        scratch_shapes=[
            pltpu.VMEM((2, DFF, D), jnp.float32),
            pltpu.VMEM((2, DFF, D), jnp.float32),
            pltpu.VMEM((2, D, DFF), jnp.float32),
            pltpu.SemaphoreType.DMA,
            pltpu.SemaphoreType.DMA,
        ],
    )
    return pl.pallas_call(
        _ffn_body,
        grid_spec=grid_spec,
        out_shape=jax.ShapeDtypeStruct((RP, D), jnp.float32),
        compiler_params=pltpu.CompilerParams(
            dimension_semantics=("arbitrary",),
            vmem_limit_bytes=100 * 1024 * 1024),
    )(blk_sched, Xg, Wx, Vx, Wo)


def _sc_mesh():
    return plsc.VectorSubcoreMesh(core_axis_name="c", subcore_axis_name="s",
                                  num_cores=2, num_subcores=16)


def _dispatch(tokens, pos0, pos1):
    def body(x_hbm, p0_hbm, p1_hbm, out_hbm, i0, i1, rows_v, sem):
        wid = lax.axis_index("s") * 2 + lax.axis_index("c")
        base = wid * TPW
        a = pltpu.async_copy(p0_hbm.at[pl.ds(base, TPW)], i0, sem)
        b = pltpu.async_copy(p1_hbm.at[pl.ds(base, TPW)], i1, sem)
        c = pltpu.async_copy(x_hbm.at[pl.ds(base, TPW)], rows_v, sem)
        a.wait()
        b.wait()
        c.wait()
        c1 = pltpu.async_copy(rows_v, out_hbm.at[i0], sem)
        c2 = pltpu.async_copy(rows_v, out_hbm.at[i1], sem)
        c1.wait()
        c2.wait()

    f = pl.kernel(
        body,
        out_type=jax.ShapeDtypeStruct((RP, D), jnp.float32),
        mesh=_sc_mesh(),
        scratch_types=[
            pltpu.VMEM((TPW,), jnp.int32),
            pltpu.VMEM((TPW,), jnp.int32),
            pltpu.VMEM((TPW, D), jnp.float32),
            pltpu.SemaphoreType.DMA,
        ],
    )
    return f(tokens, pos0, pos1)


def _combine(Og, pos0, pos1, w0x, w1x):
    def body(og_hbm, p0_hbm, p1_hbm, w0_hbm, w1_hbm, out_hbm,
             i0, i1, av, bv, wv0, wv1, sem):
        wid = lax.axis_index("s") * 2 + lax.axis_index("c")
        base = wid * TPW
        a = pltpu.async_copy(p0_hbm.at[pl.ds(base, TPW)], i0, sem)
        b = pltpu.async_copy(p1_hbm.at[pl.ds(base, TPW)], i1, sem)
        a.wait()
        b.wait()
        c1 = pltpu.async_copy(og_hbm.at[i0], av, sem)
        c2 = pltpu.async_copy(og_hbm.at[i1], bv, sem)
        pltpu.sync_copy(w0_hbm.at[pl.ds(base, TPW)], wv0)
        pltpu.sync_copy(w1_hbm.at[pl.ds(base, TPW)], wv1)
        c1.wait()
        c2.wait()

        def row(r, _):
            wa = wv0[r, :]
            wb = wv1[r, :]
            for c in range(D // 16):
                s = pl.ds(c * 16, 16)
                av[r, s] = av[r, s] * wa + bv[r, s] * wb
            return 0

        lax.fori_loop(0, TPW, row, 0)
        pltpu.sync_copy(av, out_hbm.at[pl.ds(base, TPW)])

    f = pl.kernel(
        body,
        out_type=jax.ShapeDtypeStruct((T, D), jnp.float32),
        mesh=_sc_mesh(),
        scratch_types=[
            pltpu.VMEM((TPW,), jnp.int32),
            pltpu.VMEM((TPW,), jnp.int32),
            pltpu.VMEM((TPW, D), jnp.float32),
            pltpu.VMEM((TPW, D), jnp.float32),
            pltpu.VMEM((TPW, 16), jnp.float32),
            pltpu.VMEM((TPW, 16), jnp.float32),
            pltpu.SemaphoreType.DMA,
        ],
    )
    return f(Og, pos0, pos1, w0x, w1x)


@functools.partial(jax.jit, static_argnames=())
def kernel(x, gate_W, gate_b, Wx, Vx, Wo):
    bsz, seq, d = x.shape
    tokens = x.reshape(-1, d)
    gb = gate_b.reshape(1, E)
    (logits, pos0c, pos1c, w0x, w1x, beid_row) = _route(tokens, gate_W, gb)
    pos0 = pos0c.reshape(T)
    pos1 = pos1c.reshape(T)
    blk_sched = beid_row[0]
    Xg = _dispatch(tokens, pos0, pos1)
    Og = _ffn(blk_sched, Xg, Wx, Vx, Wo)
    out = _combine(Og, pos0, pos1, w0x, w1x)
    return out.reshape(bsz, seq, d), logits
